# baseline (speedup 1.0000x reference)
import functools

import jax
import jax.numpy as jnp
from jax import lax
from jax.experimental import pallas as pl
from jax.experimental.pallas import tpu as pltpu
from jax.experimental.pallas import tpu_sc as plsc

_B = 16384
_D = 32
_CHUNK = 128


@functools.lru_cache(maxsize=None)
def _build():
    info = plsc.get_sparse_core_info()
    nc, ns = info.num_cores, info.num_subcores
    nw = nc * ns
    bw = _B // nw
    nchunk = bw // _CHUNK
    mesh = plsc.VectorSubcoreMesh(core_axis_name="c", subcore_axis_name="s")

    @functools.partial(
        pl.kernel,
        mesh=mesh,
        compiler_params=pltpu.CompilerParams(use_tc_tiling_on_sc=False),
        out_type=(
            jax.ShapeDtypeStruct((_B, _D), jnp.float32),
            jax.ShapeDtypeStruct((_B, _D), jnp.float32),
            jax.ShapeDtypeStruct((_B, _D), jnp.float32),
        ),
        scratch_types=[
            pltpu.VMEM((bw,), jnp.int32),
            pltpu.VMEM((bw,), jnp.int32),
            pltpu.VMEM((bw,), jnp.int32),
            pltpu.VMEM((bw, _D), jnp.float32),
            pltpu.VMEM((bw, _D), jnp.float32),
            pltpu.VMEM((bw, _D), jnp.float32),
            pltpu.SemaphoreType.DMA,
        ],
    )
    def k(head_hbm, rel_hbm, tail_hbm, ent_hbm, relw_hbm,
          out_h, out_r, out_t,
          hidx, ridx, tidx, hrow, rrow, trow, sem):
        wid = lax.axis_index("s") * nc + lax.axis_index("c")
        base = wid * bw
        pltpu.sync_copy(head_hbm.at[pl.ds(base, bw)], hidx)
        pltpu.sync_copy(rel_hbm.at[pl.ds(base, bw)], ridx)
        pltpu.sync_copy(tail_hbm.at[pl.ds(base, bw)], tidx)
        copies = []
        for j in range(nchunk):
            s = pl.ds(j * _CHUNK, _CHUNK)
            copies.append(pltpu.async_copy(
                ent_hbm.at[hidx.at[s]], hrow.at[s], sem))
            copies.append(pltpu.async_copy(
                relw_hbm.at[ridx.at[s]], rrow.at[s], sem))
            copies.append(pltpu.async_copy(
                ent_hbm.at[tidx.at[s]], trow.at[s], sem))
        for c in copies:
            c.wait()
        pltpu.sync_copy(hrow, out_h.at[pl.ds(base, bw)])
        pltpu.sync_copy(rrow, out_r.at[pl.ds(base, bw)])
        pltpu.sync_copy(trow, out_t.at[pl.ds(base, bw)])

    return k


def kernel(head, rel, tail, entity_table, relation_table):
    return _build()(head, rel, tail, entity_table, relation_table)
